# Initial kernel scaffold; baseline (speedup 1.0000x reference)
import jax
import jax.numpy as jnp
from jax.experimental import pallas as pl


def kernel(x, conv1_w, conv1_b, conv2_w, conv2_b, incep1_fused_w, incep1_fused_b, incep1_b3_2_w, incep1_b3_2_b, incep1_b4_2_w, incep1_b4_2_b, incep1_b4_3_w, incep1_b4_3_b, incep2_fused_w, incep2_fused_b, incep2_b3_2_w, incep2_b3_2_b, incep2_b4_2_w, incep2_b4_2_b, incep2_b4_3_w, incep2_b4_3_b, fc_w, fc_b):
    raise NotImplementedError("write your pallas kernel here")


# speedup vs baseline: 46.9699x; 46.9699x over previous
import jax
import jax.numpy as jnp
from jax.experimental import pallas as pl
from jax.experimental.pallas import tpu as pltpu

_B = 64


def _patches(x4, kh, kw):
    C, H, W, B = x4.shape
    Ho, Wo = H - kh + 1, W - kw + 1
    rows = []
    for i in range(kh):
        for j in range(kw):
            s = x4[:, i:i + Ho, j:j + Wo, :]
            rows.append(s.reshape(C, Ho * Wo * B))
    return jnp.concatenate(rows, axis=0), Ho, Wo


def _conv(x4, w, b, kh, kw, pad):
    if pad:
        x4 = jnp.pad(x4, ((0, 0), (pad, pad), (pad, pad), (0, 0)))
    p, Ho, Wo = _patches(x4, kh, kw)
    r = jnp.dot(w, p, preferred_element_type=jnp.float32) + b
    return r, Ho, Wo


def _pool2_relu(r2, C, Ho, Wo, B):
    r = r2.reshape(C, Ho, Wo // 2, 2, B)
    r = jnp.max(r, axis=3)
    r = r.reshape(C, Ho // 2, 2, Wo // 2, B)
    r = jnp.max(r, axis=2)
    return jnp.maximum(r, 0.0).astype(jnp.bfloat16)


def _inception(h4, fw, fb, b3w, b3b, b42w, b42b, b43w, b43b):
    C, H, W, B = h4.shape
    r, _, _ = _conv(h4, fw, fb, 3, 3, 1)
    act = r.astype(jnp.bfloat16)
    b12 = act[0:40]
    b3 = act[40:56].reshape(16, H, W, B)
    b4 = act[56:72].reshape(16, H, W, B)
    b3r, _, _ = _conv(b3, b3w, b3b, 5, 5, 2)
    b4r, _, _ = _conv(b4, b42w, b42b, 3, 3, 1)
    b4 = b4r.astype(jnp.bfloat16).reshape(24, H, W, B)
    b4r, _, _ = _conv(b4, b43w, b43b, 3, 3, 1)
    return jnp.concatenate(
        [b12, b3r.astype(jnp.bfloat16), b4r.astype(jnp.bfloat16)], axis=0)


def _net_kernel(x_ref, c1w, c1b,
                i1fw, i1fb, i1b3w, i1b3b, i1b42w, i1b42b, i1b43w, i1b43b,
                c2w, c2b,
                i2fw, i2fb, i2b3w, i2b3b, i2b42w, i2b42b, i2b43w, i2b43b,
                fcw, fcb, o_ref):
    B = x_ref.shape[-1]
    x4 = x_ref[...].reshape(1, 28, 28, B)

    r, Ho, Wo = _conv(x4, c1w[...], c1b[...], 5, 5, 0)
    h1 = _pool2_relu(r, 10, Ho, Wo, B)

    h2m = _inception(h1, i1fw[...], i1fb[...], i1b3w[...], i1b3b[...],
                     i1b42w[...], i1b42b[...], i1b43w[...], i1b43b[...])
    h2 = h2m.reshape(88, 12, 12, B)

    w2 = c2w[...]
    b2 = c2b[...]
    h3_rows = []
    for oh in range(4):
        rows = []
        for i in range(5):
            for j in range(5):
                s = h2[:, 2 * oh + i:2 * oh + i + 2, j:j + 8, :]
                rows.append(s.reshape(88, 2 * 8 * B))
        p = jnp.concatenate(rows, axis=0)
        rr = jnp.dot(w2, p, preferred_element_type=jnp.float32) + b2
        h3_rows.append(_pool2_relu(rr, 20, 2, 8, B))
    h3 = jnp.concatenate(h3_rows, axis=1)

    h4m = _inception(h3, i2fw[...], i2fb[...], i2b3w[...], i2b3b[...],
                     i2b42w[...], i2b42b[...], i2b43w[...], i2b43b[...])

    out = fcb[...].astype(jnp.float32) * jnp.ones((10, B), jnp.float32)
    for k in range(16):
        out = out + jnp.dot(fcw[k], h4m[:, k * B:(k + 1) * B],
                            preferred_element_type=jnp.float32)
    o_ref[...] = out


def kernel(x, conv1_w, conv1_b, conv2_w, conv2_b,
           incep1_fused_w, incep1_fused_b, incep1_b3_2_w, incep1_b3_2_b,
           incep1_b4_2_w, incep1_b4_2_b, incep1_b4_3_w, incep1_b4_3_b,
           incep2_fused_w, incep2_fused_b, incep2_b3_2_w, incep2_b3_2_b,
           incep2_b4_2_w, incep2_b4_2_b, incep2_b4_3_w, incep2_b4_3_b,
           fc_w, fc_b):
    N = x.shape[0]
    B = _B
    Np = ((N + B - 1) // B) * B
    xb = x.astype(jnp.bfloat16).reshape(N, 28, 28)
    if Np != N:
        xb = jnp.pad(xb, ((0, Np - N), (0, 0), (0, 0)))
    xt = jnp.transpose(xb.reshape(Np // B, B, 28, 28), (0, 2, 3, 1))
    fcp = jnp.transpose(fc_w.reshape(10, 88, 16), (2, 0, 1))

    def full(a):
        return pl.BlockSpec(a.shape, lambda i: (0,) * a.ndim)

    ws = [conv1_w, conv1_b,
          incep1_fused_w, incep1_fused_b, incep1_b3_2_w, incep1_b3_2_b,
          incep1_b4_2_w, incep1_b4_2_b, incep1_b4_3_w, incep1_b4_3_b,
          conv2_w, conv2_b,
          incep2_fused_w, incep2_fused_b, incep2_b3_2_w, incep2_b3_2_b,
          incep2_b4_2_w, incep2_b4_2_b, incep2_b4_3_w, incep2_b4_3_b,
          fcp, fc_b]

    out_b = pl.pallas_call(
        _net_kernel,
        out_shape=jax.ShapeDtypeStruct((Np // B, 10, B), jnp.float32),
        grid=(Np // B,),
        in_specs=[pl.BlockSpec((None, 28, 28, B), lambda i: (i, 0, 0, 0))] +
                 [full(a) for a in ws],
        out_specs=pl.BlockSpec((None, 10, B), lambda i: (i, 0, 0)),
        compiler_params=pltpu.CompilerParams(
            dimension_semantics=("parallel",)),
    )(xt, *ws)
    return jnp.transpose(out_b, (0, 2, 1)).reshape(Np, 10)[:N]
